# baseline (speedup 1.0000x reference)
import functools

import jax
import jax.numpy as jnp
from jax import lax
from jax.experimental import pallas as pl
from jax.experimental.pallas import tpu as pltpu
from jax.experimental.pallas import tpu_sc as plsc

N = 10000
NP = 10240
E = 160000
D = 256
DH = 128
EMB = 128
NC = 2
NS = 16
RT = NP // NS
KE = 400
ET = E // NS
BP = 51200
PT = BP // NS

_f32 = jnp.float32
_bf16 = jnp.bfloat16


def _sc_mesh():
  return plsc.VectorSubcoreMesh(
      core_axis_name="c", subcore_axis_name="s", num_cores=NC,
      num_subcores=NS)


def _conv_body(do_counts, y_hbm, srcoff_hbm, dst_hbm, *rest):
  if do_counts:
    (s_hbm, cnt_hbm, idxs, idxd, rows0, rows1, zb, ones, zbc, shared,
     cshared, sem0, sem1) = rest
  else:
    s_hbm, idxs, idxd, rows0, rows1, zb, shared, sem0, sem1 = rest
  c = lax.axis_index("c")
  s = lax.axis_index("s")
  base = s * RT
  nch = ET // KE

  for i in range(16):
    for j in range(DH // 32):
      zb[i, pl.ds(j * 32, 32)] = jnp.zeros((32,), _bf16)
  if do_counts:
    for i in range(16):
      zbc[i, :] = jnp.zeros((16,), _f32)
    for i in range(KE):
      ones[i, :] = jnp.full((16,), 1.0, _f32)

  def zrow(k, carry):
    pltpu.sync_copy(zb, shared.at[pl.ds(base + k * 16, 16)])
    return carry
  lax.fori_loop(0, RT // 16, zrow, 0)
  if do_counts:
    def zrowc(k, carry):
      pltpu.sync_copy(zbc, cshared.at[pl.ds(base + k * 16, 16)])
      return carry
    lax.fori_loop(0, RT // 16, zrowc, 0)
  pltpu.sync_copy(dst_hbm.at[pl.ds(s * nch, nch)], idxd)
  pltpu.sync_copy(srcoff_hbm.at[pl.ds((c * NS + s) * nch, nch)], idxs)
  plsc.subcore_barrier()

  pltpu.async_copy(y_hbm.at[idxs.at[0]], rows0, sem0)
  def pair2(j, carry):
    a = 2 * j
    pltpu.async_copy(y_hbm.at[idxs.at[a + 1]], rows1, sem1)
    pltpu.make_async_copy(y_hbm.at[idxs.at[a]], rows0, sem0).wait()
    pltpu.sync_copy(rows0, shared.at[idxd.at[a]], add=True)
    pltpu.async_copy(y_hbm.at[idxs.at[a + 2]], rows0, sem0)
    pltpu.make_async_copy(y_hbm.at[idxs.at[a + 1]], rows1, sem1).wait()
    pltpu.sync_copy(rows1, shared.at[idxd.at[a + 1]], add=True)
    return carry
  lax.fori_loop(0, (nch - 1) // 2, pair2, 0)
  pltpu.make_async_copy(y_hbm.at[idxs.at[nch - 1]], rows0, sem0).wait()
  pltpu.sync_copy(rows0, shared.at[idxd.at[nch - 1]], add=True)

  if do_counts:
    def cchunk(k, carry):
      pltpu.sync_copy(ones, cshared.at[idxd.at[k]], add=True)
      return carry
    lax.fori_loop(0, nch, cchunk, 0)

  plsc.subcore_barrier()
  pltpu.sync_copy(shared.at[pl.ds(base, RT)],
                  s_hbm.at[pl.ds(c * NP + base, RT)])
  if do_counts:
    pltpu.sync_copy(cshared.at[pl.ds(base, RT)],
                    cnt_hbm.at[pl.ds(c * NP + base, RT)])


def _make_conv(do_counts):
  nch = ET // KE
  out_type = [jax.ShapeDtypeStruct((NC * NP, DH), _bf16)]
  scratch = [
      pltpu.VMEM((nch, KE), jnp.int32),
      pltpu.VMEM((nch, KE), jnp.int32),
      pltpu.VMEM((KE, DH), _bf16),
      pltpu.VMEM((KE, DH), _bf16),
      pltpu.VMEM((16, DH), _bf16),
  ]
  if do_counts:
    out_type.append(jax.ShapeDtypeStruct((NC * NP, 16), _f32))
    scratch += [
        pltpu.VMEM((KE, 16), _f32),
        pltpu.VMEM((16, 16), _f32),
    ]
  scratch.append(pltpu.VMEM_SHARED((NP, DH), _bf16))
  if do_counts:
    scratch.append(pltpu.VMEM_SHARED((NP, 16), _f32))
  scratch += [pltpu.SemaphoreType.DMA, pltpu.SemaphoreType.DMA]
  return pl.kernel(
      functools.partial(_conv_body, do_counts),
      out_type=out_type, mesh=_sc_mesh(), scratch_types=scratch,
      compiler_params=pltpu.CompilerParams(use_tc_tiling_on_sc=False),
      name="sc_segsum" + ("_cnt" if do_counts else ""))


_make_conv = functools.cache(_make_conv)


def _pair_body(au_hbm, am_hbm, uid_hbm, mid_hbm, g_hbm, idxu, idxm, rows,
               sem):
  c = lax.axis_index("c")
  s = lax.axis_index("s")
  nch = PT // KE
  pbase = c * BP + s * PT
  pltpu.sync_copy(uid_hbm.at[pl.ds((c * NS + s) * nch, nch)], idxu)
  pltpu.sync_copy(mid_hbm.at[pl.ds((c * NS + s) * nch, nch)], idxm)
  def chunk(k, carry):
    off = pbase + k * KE
    pltpu.async_copy(au_hbm.at[idxu.at[k]], rows, sem).wait()
    pltpu.async_copy(am_hbm.at[idxm.at[k]], rows, sem, add=True).wait()
    pltpu.sync_copy(rows, g_hbm.at[pl.ds(off, KE)])
    return carry
  lax.fori_loop(0, nch, chunk, 0)


@functools.cache
def _make_pair():
  nch = PT // KE
  return pl.kernel(
      _pair_body,
      out_type=[jax.ShapeDtypeStruct((NC * BP, DH), _bf16)],
      mesh=_sc_mesh(),
      scratch_types=[
          pltpu.VMEM((nch, KE), jnp.int32),
          pltpu.VMEM((nch, KE), jnp.int32),
          pltpu.VMEM((KE, DH), _bf16),
          pltpu.SemaphoreType.DMA,
      ],
      compiler_params=pltpu.CompilerParams(use_tc_tiling_on_sc=False),
      name="sc_pair_gather")


_BN = 1024


def _stack_halves(y):
  return jnp.concatenate([y[None, :, :DH], y[None, :, DH:]], axis=0)


def _mm_halves_body(x_ref, w_ref, o_ref):
  o_ref[...] = jnp.dot(x_ref[...], w_ref[...],
                       preferred_element_type=_f32).astype(_bf16)


def _mm_halves(x, w):
  gi = NP // _BN
  return pl.pallas_call(
      _mm_halves_body,
      grid=(gi, NC),
      in_specs=[
          pl.BlockSpec((_BN, D), lambda i, j: (i, 0)),
          pl.BlockSpec((D, DH), lambda i, j: (0, j)),
      ],
      out_specs=pl.BlockSpec((_BN, DH), lambda i, j, gi=gi: (j * gi + i, 0)),
      out_shape=jax.ShapeDtypeStruct((NC * NP, DH), _bf16),
  )(x, w)


def _epilogue_body(relu, s_ref, c_ref, x_ref, w_ref, b_ref, o_ref):
  sv = s_ref[...]
  cat = jnp.concatenate([sv[0], sv[1]], axis=1).astype(_f32)
  cv = c_ref[...]
  cnt = (cv[0][:, :1] + cv[1][:, :1]) * 0.5
  inv = 1.0 / jnp.maximum(cnt, 1.0)
  h = cat * inv + b_ref[...] + jnp.dot(x_ref[...], w_ref[...],
                                       preferred_element_type=_f32)
  if relu:
    h = jnp.maximum(h, 0.0)
  o_ref[...] = h


def _epilogue(s_flat, cnt_flat, x_dst, wr, b, relu):
  s2 = s_flat.reshape(NC, NP, DH)
  c2 = cnt_flat.reshape(NC, NP, 16)
  return pl.pallas_call(
      functools.partial(_epilogue_body, relu),
      grid=(NP // _BN,),
      in_specs=[
          pl.BlockSpec((NC, _BN, DH), lambda i: (0, i, 0)),
          pl.BlockSpec((NC, _BN, 16), lambda i: (0, i, 0)),
          pl.BlockSpec((_BN, D), lambda i: (i, 0)),
          pl.BlockSpec((D, D), lambda i: (0, 0)),
          pl.BlockSpec((1, D), lambda i: (0, 0)),
      ],
      out_specs=pl.BlockSpec((_BN, D), lambda i: (i, 0)),
      out_shape=jax.ShapeDtypeStruct((NP, D), _f32),
  )(s2, c2, x_dst, wr, b.reshape(1, D))


def _head_a_body(h_ref, t_ref, wp_ref, bp_ref, c_ref, o_ref):
  wp = wp_ref[...]
  pu = (jnp.dot(h_ref[...], wp[:D], preferred_element_type=_f32)
        + jnp.dot(t_ref[...], wp[D:], preferred_element_type=_f32)
        + bp_ref[...])
  o_ref[...] = jnp.dot(pu, c_ref[...],
                       preferred_element_type=_f32).astype(_bf16)


def _head_a(h, table, wp, bp, c_mat):
  gi = NP // _BN
  return pl.pallas_call(
      _head_a_body,
      grid=(gi, NC),
      in_specs=[
          pl.BlockSpec((_BN, D), lambda i, j: (i, 0)),
          pl.BlockSpec((_BN, EMB), lambda i, j: (i, 0)),
          pl.BlockSpec((D + EMB, D), lambda i, j: (0, 0)),
          pl.BlockSpec((1, D), lambda i, j: (0, 0)),
          pl.BlockSpec((D, DH), lambda i, j: (0, j)),
      ],
      out_specs=pl.BlockSpec((_BN, DH), lambda i, j, gi=gi: (j * gi + i, 0)),
      out_shape=jax.ShapeDtypeStruct((NC * NP, DH), _bf16),
  )(h, table, wp, bp.reshape(1, D), c_mat)


_BNP = 1024


def _final_body(g_ref, w_ref, b1_ref, b2_ref, o_ref):
  g = g_ref[...]
  cat = jnp.concatenate([g[0], g[1]], axis=1).astype(_f32)
  hid = jnp.maximum(cat + b1_ref[...], 0.0)
  o_ref[...] = jnp.dot(hid, w_ref[...],
                       preferred_element_type=_f32) + b2_ref[...]


def _final(g_flat, wc2p, bc1, bc2p):
  g2 = g_flat.reshape(NC, BP, DH)
  return pl.pallas_call(
      _final_body,
      grid=(BP // _BNP,),
      in_specs=[
          pl.BlockSpec((NC, _BNP, DH), lambda i: (0, i, 0)),
          pl.BlockSpec((D, EMB), lambda i: (0, 0)),
          pl.BlockSpec((1, D), lambda i: (0, 0)),
          pl.BlockSpec((1, EMB), lambda i: (0, 0)),
      ],
      out_specs=pl.BlockSpec((_BNP, EMB), lambda i: (i, 0)),
      out_shape=jax.ShapeDtypeStruct((BP, EMB), _f32),
  )(g2, wc2p, bc1.reshape(1, D), bc2p.reshape(1, EMB))


def kernel(x_user, x_movie, edge_index_um, edge_index_mu, user_movie_pairs,
           user_table, movie_table,
           W1l_um, b1_um, W1r_um, W1l_mu, b1_mu, W1r_mu,
           W2l_um, b2_um, W2r_um, W2l_mu, b2_mu, W2r_mu,
           Wp, bp, Wc1, bc1, Wc2, bc2):
  i32 = jnp.int32
  src_um = edge_index_um[0].astype(i32)
  dst_um = edge_index_um[1].astype(i32)
  src_mu = edge_index_mu[0].astype(i32)
  dst_mu = edge_index_mu[1].astype(i32)
  srcoff_um = jnp.concatenate([src_um, src_um + NP]).reshape(-1, KE)
  srcoff_mu = jnp.concatenate([src_mu, src_mu + NP]).reshape(-1, KE)
  dst2_um = dst_um.reshape(-1, KE)
  dst2_mu = dst_mu.reshape(-1, KE)
  zpad = ((0, NP - N), (0, 0))
  x_user = jnp.pad(x_user, zpad)
  x_movie = jnp.pad(x_movie, zpad)
  user_table = jnp.pad(user_table, zpad)
  movie_table = jnp.pad(movie_table, zpad)

  conv_cnt = _make_conv(True)
  conv = _make_conv(False)

  y1u = _mm_halves(x_user, W1l_um)
  y1m = _mm_halves(x_movie, W1l_mu)
  s1m, cnt_um = conv_cnt(y1u, srcoff_um, dst2_um)
  s1u, cnt_mu = conv_cnt(y1m, srcoff_mu, dst2_mu)
  h_movie1 = _epilogue(s1m, cnt_um, x_movie, W1r_um, b1_um, True)
  h_user1 = _epilogue(s1u, cnt_mu, x_user, W1r_mu, b1_mu, True)

  y2u = _mm_halves(h_user1, W2l_um)
  y2m = _mm_halves(h_movie1, W2l_mu)
  (s2m,) = conv(y2u, srcoff_um, dst2_um)
  (s2u,) = conv(y2m, srcoff_mu, dst2_mu)
  h_movie = _epilogue(s2m, cnt_um, h_movie1, W2r_um, b2_um, False)
  h_user = _epilogue(s2u, cnt_mu, h_user1, W2r_mu, b2_mu, False)

  a_user = _head_a(h_user, user_table, Wp, bp, Wc1[:D])
  a_movie = _head_a(h_movie, movie_table, Wp, bp, Wc1[D:])

  uid = user_movie_pairs[0].astype(i32)
  mid = user_movie_pairs[1].astype(i32)
  npad = BP - uid.shape[0]
  pad = (jnp.arange(npad, dtype=i32) * 37) % N
  uid_p = jnp.concatenate([uid, pad])
  mid_p = jnp.concatenate([mid, pad])
  uidoff = jnp.concatenate([uid_p, uid_p + NP]).reshape(-1, KE)
  midoff = jnp.concatenate([mid_p, mid_p + NP]).reshape(-1, KE)
  (g,) = _make_pair()(a_user, a_movie, uidoff, midoff)

  wc2p = jnp.zeros((D, EMB), _f32).at[:, :5].set(Wc2)
  bc2p = jnp.zeros((EMB,), _f32).at[:5].set(bc2)
  out = _final(g, wc2p, bc1, bc2p)
  return out[:user_movie_pairs.shape[1], :5]


# speedup vs baseline: 1.1273x; 1.1273x over previous
import functools

import jax
import jax.numpy as jnp
from jax import lax
from jax.experimental import pallas as pl
from jax.experimental.pallas import tpu as pltpu
from jax.experimental.pallas import tpu_sc as plsc

N = 10000
NP = 10240
E = 160000
D = 256
DH = 128
EMB = 128
NC = 2
NS = 16
RT = NP // NS
KE = 400
ET = E // NS
BP = 51200
PT = BP // NS

_f32 = jnp.float32
_bf16 = jnp.bfloat16


def _sc_mesh():
  return plsc.VectorSubcoreMesh(
      core_axis_name="c", subcore_axis_name="s", num_cores=NC,
      num_subcores=NS)


def _conv_body(do_counts, y_hbm, srcoff_hbm, dst_hbm, *rest):
  if do_counts:
    (s_hbm, cnt_hbm, idxs, idxd, rows0, rows1, zb, ones, zbc, shared,
     cshared, sem0, sem1) = rest
  else:
    s_hbm, idxs, idxd, rows0, rows1, zb, shared, sem0, sem1 = rest
  c = lax.axis_index("c")
  s = lax.axis_index("s")
  base = s * RT
  nch = ET // KE

  for i in range(16):
    for j in range(DH // 32):
      zb[i, pl.ds(j * 32, 32)] = jnp.zeros((32,), _bf16)
  if do_counts:
    for i in range(16):
      zbc[i, :] = jnp.zeros((16,), _f32)
    for i in range(KE):
      ones[i, :] = jnp.full((16,), 1.0, _f32)

  def zrow(k, carry):
    pltpu.sync_copy(zb, shared.at[pl.ds(base + k * 16, 16)])
    return carry
  lax.fori_loop(0, RT // 16, zrow, 0)
  if do_counts:
    def zrowc(k, carry):
      pltpu.sync_copy(zbc, cshared.at[pl.ds(base + k * 16, 16)])
      return carry
    lax.fori_loop(0, RT // 16, zrowc, 0)
  pltpu.sync_copy(dst_hbm.at[pl.ds(s * nch, nch)], idxd)
  pltpu.sync_copy(srcoff_hbm.at[pl.ds((c * NS + s) * nch, nch)], idxs)
  plsc.subcore_barrier()

  pltpu.async_copy(y_hbm.at[idxs.at[0]], rows0, sem0)
  def pair2(j, carry):
    a = 2 * j
    pltpu.async_copy(y_hbm.at[idxs.at[a + 1]], rows1, sem1)
    pltpu.make_async_copy(y_hbm.at[idxs.at[a]], rows0, sem0).wait()
    pltpu.sync_copy(rows0, shared.at[idxd.at[a]], add=True)
    pltpu.async_copy(y_hbm.at[idxs.at[a + 2]], rows0, sem0)
    pltpu.make_async_copy(y_hbm.at[idxs.at[a + 1]], rows1, sem1).wait()
    pltpu.sync_copy(rows1, shared.at[idxd.at[a + 1]], add=True)
    return carry
  lax.fori_loop(0, (nch - 1) // 2, pair2, 0)
  pltpu.make_async_copy(y_hbm.at[idxs.at[nch - 1]], rows0, sem0).wait()
  pltpu.sync_copy(rows0, shared.at[idxd.at[nch - 1]], add=True)

  if do_counts:
    def cchunk(k, carry):
      pltpu.sync_copy(ones, cshared.at[idxd.at[k]], add=True)
      return carry
    lax.fori_loop(0, nch, cchunk, 0)

  plsc.subcore_barrier()
  pltpu.sync_copy(shared.at[pl.ds(base, RT)],
                  s_hbm.at[pl.ds(c * NP + base, RT)])
  if do_counts:
    pltpu.sync_copy(cshared.at[pl.ds(base, RT)],
                    cnt_hbm.at[pl.ds(c * NP + base, RT)])


def _make_conv(do_counts):
  nch = ET // KE
  out_type = [jax.ShapeDtypeStruct((NC * NP, DH), _bf16)]
  scratch = [
      pltpu.VMEM((nch, KE), jnp.int32),
      pltpu.VMEM((nch, KE), jnp.int32),
      pltpu.VMEM((KE, DH), _bf16),
      pltpu.VMEM((KE, DH), _bf16),
      pltpu.VMEM((16, DH), _bf16),
  ]
  if do_counts:
    out_type.append(jax.ShapeDtypeStruct((NC * NP, 16), _f32))
    scratch += [
        pltpu.VMEM((KE, 16), _f32),
        pltpu.VMEM((16, 16), _f32),
    ]
  scratch.append(pltpu.VMEM_SHARED((NP, DH), _bf16))
  if do_counts:
    scratch.append(pltpu.VMEM_SHARED((NP, 16), _f32))
  scratch += [pltpu.SemaphoreType.DMA, pltpu.SemaphoreType.DMA]
  return pl.kernel(
      functools.partial(_conv_body, do_counts),
      out_type=out_type, mesh=_sc_mesh(), scratch_types=scratch,
      compiler_params=pltpu.CompilerParams(use_tc_tiling_on_sc=False),
      name="sc_segsum" + ("_cnt" if do_counts else ""))


_make_conv = functools.cache(_make_conv)


def _pair_body(au_hbm, am_hbm, uid_hbm, mid_hbm, g_hbm, idxu, idxm, rows,
               sem):
  c = lax.axis_index("c")
  s = lax.axis_index("s")
  pbase = c * BP + s * PT
  def chunk(k, carry):
    off = pbase + k * KE
    pltpu.sync_copy(uid_hbm.at[pl.ds(off, KE)], idxu)
    pltpu.sync_copy(mid_hbm.at[pl.ds(off, KE)], idxm)
    pltpu.async_copy(au_hbm.at[idxu], rows, sem).wait()
    pltpu.async_copy(am_hbm.at[idxm], rows, sem, add=True).wait()
    pltpu.sync_copy(rows, g_hbm.at[pl.ds(off, KE)])
    return carry
  lax.fori_loop(0, PT // KE, chunk, 0)


@functools.cache
def _make_pair():
  return pl.kernel(
      _pair_body,
      out_type=[jax.ShapeDtypeStruct((NC * BP, DH), _f32)],
      mesh=_sc_mesh(),
      scratch_types=[
          pltpu.VMEM((KE,), jnp.int32),
          pltpu.VMEM((KE,), jnp.int32),
          pltpu.VMEM((KE, DH), _f32),
          pltpu.SemaphoreType.DMA,
      ],
      name="sc_pair_gather")


_BN = 1024


def _stack_halves(y):
  return jnp.concatenate([y[None, :, :DH], y[None, :, DH:]], axis=0)


def _mm_halves_body(x_ref, w_ref, o_ref):
  o_ref[...] = jnp.dot(x_ref[...], w_ref[...],
                       preferred_element_type=_f32).astype(_bf16)


def _mm_halves(x, w):
  gi = NP // _BN
  return pl.pallas_call(
      _mm_halves_body,
      grid=(gi, NC),
      in_specs=[
          pl.BlockSpec((_BN, D), lambda i, j: (i, 0)),
          pl.BlockSpec((D, DH), lambda i, j: (0, j)),
      ],
      out_specs=pl.BlockSpec((_BN, DH), lambda i, j, gi=gi: (j * gi + i, 0)),
      out_shape=jax.ShapeDtypeStruct((NC * NP, DH), _bf16),
  )(x, w)


def _epilogue_body(relu, s_ref, c_ref, x_ref, w_ref, b_ref, o_ref):
  sv = s_ref[...]
  cat = jnp.concatenate([sv[0], sv[1]], axis=1).astype(_f32)
  cv = c_ref[...]
  cnt = (cv[0][:, :1] + cv[1][:, :1]) * 0.5
  inv = 1.0 / jnp.maximum(cnt, 1.0)
  h = cat * inv + b_ref[...] + jnp.dot(x_ref[...], w_ref[...],
                                       preferred_element_type=_f32)
  if relu:
    h = jnp.maximum(h, 0.0)
  o_ref[...] = h


def _epilogue(s_flat, cnt_flat, x_dst, wr, b, relu):
  s2 = s_flat.reshape(NC, NP, DH)
  c2 = cnt_flat.reshape(NC, NP, 16)
  return pl.pallas_call(
      functools.partial(_epilogue_body, relu),
      grid=(NP // _BN,),
      in_specs=[
          pl.BlockSpec((NC, _BN, DH), lambda i: (0, i, 0)),
          pl.BlockSpec((NC, _BN, 16), lambda i: (0, i, 0)),
          pl.BlockSpec((_BN, D), lambda i: (i, 0)),
          pl.BlockSpec((D, D), lambda i: (0, 0)),
          pl.BlockSpec((1, D), lambda i: (0, 0)),
      ],
      out_specs=pl.BlockSpec((_BN, D), lambda i: (i, 0)),
      out_shape=jax.ShapeDtypeStruct((NP, D), _f32),
  )(s2, c2, x_dst, wr, b.reshape(1, D))


def _head_a_body(h_ref, t_ref, wp_ref, bp_ref, c_ref, o_ref):
  wp = wp_ref[...]
  pu = (jnp.dot(h_ref[...], wp[:D], preferred_element_type=_f32)
        + jnp.dot(t_ref[...], wp[D:], preferred_element_type=_f32)
        + bp_ref[...])
  o_ref[...] = jnp.dot(pu, c_ref[...], preferred_element_type=_f32)


def _head_a(h, table, wp, bp, c_mat):
  gi = NP // _BN
  return pl.pallas_call(
      _head_a_body,
      grid=(gi, NC),
      in_specs=[
          pl.BlockSpec((_BN, D), lambda i, j: (i, 0)),
          pl.BlockSpec((_BN, EMB), lambda i, j: (i, 0)),
          pl.BlockSpec((D + EMB, D), lambda i, j: (0, 0)),
          pl.BlockSpec((1, D), lambda i, j: (0, 0)),
          pl.BlockSpec((D, DH), lambda i, j: (0, j)),
      ],
      out_specs=pl.BlockSpec((_BN, DH), lambda i, j, gi=gi: (j * gi + i, 0)),
      out_shape=jax.ShapeDtypeStruct((NC * NP, DH), _f32),
  )(h, table, wp, bp.reshape(1, D), c_mat)


_BNP = 1024


def _final_body(g_ref, w_ref, b1_ref, b2_ref, o_ref):
  g = g_ref[...]
  cat = jnp.concatenate([g[0], g[1]], axis=1)
  hid = jnp.maximum(cat + b1_ref[...], 0.0)
  o_ref[...] = jnp.dot(hid, w_ref[...],
                       preferred_element_type=_f32) + b2_ref[...]


def _final(g_flat, wc2p, bc1, bc2p):
  g2 = g_flat.reshape(NC, BP, DH)
  return pl.pallas_call(
      _final_body,
      grid=(BP // _BNP,),
      in_specs=[
          pl.BlockSpec((NC, _BNP, DH), lambda i: (0, i, 0)),
          pl.BlockSpec((D, EMB), lambda i: (0, 0)),
          pl.BlockSpec((1, D), lambda i: (0, 0)),
          pl.BlockSpec((1, EMB), lambda i: (0, 0)),
      ],
      out_specs=pl.BlockSpec((_BNP, EMB), lambda i: (i, 0)),
      out_shape=jax.ShapeDtypeStruct((BP, EMB), _f32),
  )(g2, wc2p, bc1.reshape(1, D), bc2p.reshape(1, EMB))


def kernel(x_user, x_movie, edge_index_um, edge_index_mu, user_movie_pairs,
           user_table, movie_table,
           W1l_um, b1_um, W1r_um, W1l_mu, b1_mu, W1r_mu,
           W2l_um, b2_um, W2r_um, W2l_mu, b2_mu, W2r_mu,
           Wp, bp, Wc1, bc1, Wc2, bc2):
  i32 = jnp.int32
  src_um = edge_index_um[0].astype(i32)
  dst_um = edge_index_um[1].astype(i32)
  src_mu = edge_index_mu[0].astype(i32)
  dst_mu = edge_index_mu[1].astype(i32)
  srcoff_um = jnp.concatenate([src_um, src_um + NP]).reshape(-1, KE)
  srcoff_mu = jnp.concatenate([src_mu, src_mu + NP]).reshape(-1, KE)
  dst2_um = dst_um.reshape(-1, KE)
  dst2_mu = dst_mu.reshape(-1, KE)
  zpad = ((0, NP - N), (0, 0))
  x_user = jnp.pad(x_user, zpad)
  x_movie = jnp.pad(x_movie, zpad)
  user_table = jnp.pad(user_table, zpad)
  movie_table = jnp.pad(movie_table, zpad)

  conv_cnt = _make_conv(True)
  conv = _make_conv(False)

  y1u = _mm_halves(x_user, W1l_um)
  y1m = _mm_halves(x_movie, W1l_mu)
  s1m, cnt_um = conv_cnt(y1u, srcoff_um, dst2_um)
  s1u, cnt_mu = conv_cnt(y1m, srcoff_mu, dst2_mu)
  h_movie1 = _epilogue(s1m, cnt_um, x_movie, W1r_um, b1_um, True)
  h_user1 = _epilogue(s1u, cnt_mu, x_user, W1r_mu, b1_mu, True)

  y2u = _mm_halves(h_user1, W2l_um)
  y2m = _mm_halves(h_movie1, W2l_mu)
  (s2m,) = conv(y2u, srcoff_um, dst2_um)
  (s2u,) = conv(y2m, srcoff_mu, dst2_mu)
  h_movie = _epilogue(s2m, cnt_um, h_movie1, W2r_um, b2_um, False)
  h_user = _epilogue(s2u, cnt_mu, h_user1, W2r_mu, b2_mu, False)

  a_user = _head_a(h_user, user_table, Wp, bp, Wc1[:D])
  a_movie = _head_a(h_movie, movie_table, Wp, bp, Wc1[D:])

  uid = user_movie_pairs[0].astype(i32)
  mid = user_movie_pairs[1].astype(i32)
  npad = BP - uid.shape[0]
  pad = (jnp.arange(npad, dtype=i32) * 37) % N
  uid_p = jnp.concatenate([uid, pad])
  mid_p = jnp.concatenate([mid, pad])
  uidoff = jnp.concatenate([uid_p, uid_p + NP])
  midoff = jnp.concatenate([mid_p, mid_p + NP])
  (g,) = _make_pair()(a_user, a_movie, uidoff, midoff)

  wc2p = jnp.zeros((D, EMB), _f32).at[:, :5].set(Wc2)
  bc2p = jnp.zeros((EMB,), _f32).at[:5].set(bc2)
  out = _final(g, wc2p, bc1, bc2p)
  return out[:user_movie_pairs.shape[1], :5]
